# baseline (speedup 1.0000x reference)
import jax
import jax.numpy as jnp
from jax.experimental import pallas as pl
from jax.experimental.pallas import tpu as pltpu

N_NODES = 10000
N_EDGES = 320000
EMB = 128
NUM_RADIAL = 6
NUM_TARGETS = 12

NMAJ = 40
NSLICE = 8
ROWS = N_EDGES // (NMAJ * NSLICE)

_ROW = (((1,), (1,)), ((), ()))
_BT = (((1,), (1,)), ((0,), (0,)))


def _stream_kernel(m_hbm, rbf_hbm, WrTb_ref, W1_ref, b1_ref, W2_ref, b2_ref,
                   W3_ref, b3_ref, Wf_ref, out_ref,
                   m_bufs, rbf_bufs, sem_m, sem_r):
    def copy_m(j, slot):
        return pltpu.make_async_copy(
            m_hbm.at[:, j], m_bufs.at[slot], sem_m.at[slot])

    def copy_r(j, slot):
        return pltpu.make_async_copy(
            rbf_hbm.at[:, j], rbf_bufs.at[slot], sem_r.at[slot])

    for b in range(2):
        copy_m(b, b).start()
        copy_r(b, b).start()

    acc = jnp.zeros((8, EMB), jnp.float32)
    for j in range(NSLICE):
        slot = j % 2
        copy_m(j, slot).wait()
        copy_r(j, slot).wait()
        w = jax.lax.dot_general(rbf_bufs[slot], WrTb_ref[...], _BT,
                                preferred_element_type=jnp.float32)
        prod = (m_bufs[slot] * w).reshape(NMAJ * ROWS // 8, 8, EMB)
        acc = acc + jnp.sum(prod, axis=0)
        if j + 2 < NSLICE:
            copy_m(j + 2, slot).start()
            copy_r(j + 2, slot).start()

    n = jnp.float32(N_NODES)
    t = jnp.sum(acc, axis=0, keepdims=True)
    t = jax.lax.dot_general(t, W1_ref[...], _ROW,
                            preferred_element_type=jnp.float32,
                            precision=jax.lax.Precision.HIGHEST) + n * b1_ref[...]
    t = jax.lax.dot_general(t, W2_ref[...], _ROW,
                            preferred_element_type=jnp.float32,
                            precision=jax.lax.Precision.HIGHEST) + n * b2_ref[...]
    t = jax.lax.dot_general(t, W3_ref[...], _ROW,
                            preferred_element_type=jnp.float32,
                            precision=jax.lax.Precision.HIGHEST) + n * b3_ref[...]
    out_ref[...] = jax.lax.dot_general(t, Wf_ref[...], _ROW,
                                       preferred_element_type=jnp.float32,
                                       precision=jax.lax.Precision.HIGHEST)


def kernel(m, rbf, edge_index, W_rbf, W1, b1, W2, b2, W3, b3, W_final):
    del edge_index
    m4 = m.reshape(NMAJ, NSLICE, ROWS, EMB)
    rbf4 = rbf.reshape(NMAJ, NSLICE, ROWS, NUM_RADIAL).transpose(0, 1, 3, 2)
    WrTb = jnp.tile(W_rbf.T[None], (NMAJ, 1, 1))
    b1r = b1.reshape(1, EMB)
    b2r = b2.reshape(1, EMB)
    b3r = b3.reshape(1, EMB)
    hbm = pl.BlockSpec(memory_space=pltpu.MemorySpace.HBM)
    vmem = pl.BlockSpec(memory_space=pltpu.MemorySpace.VMEM)
    return pl.pallas_call(
        _stream_kernel,
        in_specs=[hbm, hbm, vmem, vmem, vmem, vmem, vmem, vmem, vmem, vmem],
        out_specs=vmem,
        out_shape=jax.ShapeDtypeStruct((1, NUM_TARGETS), jnp.float32),
        scratch_shapes=[
            pltpu.VMEM((2, NMAJ, ROWS, EMB), jnp.float32),
            pltpu.VMEM((2, NMAJ, NUM_RADIAL, ROWS), jnp.float32),
            pltpu.SemaphoreType.DMA((2,)),
            pltpu.SemaphoreType.DMA((2,)),
        ],
    )(m4, rbf4, WrTb, W1, b1r, W2, b2r, W3, b3r, W_final)
